# baseline (speedup 1.0000x reference)
import jax
import jax.numpy as jnp
from jax import lax
from jax.experimental import pallas as pl
from jax.experimental.pallas import tpu as pltpu
from jax.experimental.pallas import tpu_sc as plsc

VOCAB = 1000000
D = 64
BATCH = 4096
SEQ = 200
N = BATCH * SEQ

NC = 2
NS = 16
NW = NC * NS
PER_W = N // NW
CHUNK = 1024
N_CHUNKS = PER_W // CHUNK


def _gather_body(table_hbm, idx_hbm, out_hbm, idx_v, rows_v, sem):
    wid = lax.axis_index("s") * NC + lax.axis_index("c")
    w_base = wid * PER_W

    @pl.loop(0, N_CHUNKS)
    def _chunk(i):
        base = w_base + i * CHUNK
        pltpu.sync_copy(idx_hbm.at[pl.ds(base, CHUNK)], idx_v)
        pltpu.async_copy(table_hbm.at[idx_v], rows_v, sem).wait()
        pltpu.sync_copy(rows_v, out_hbm.at[pl.ds(base, CHUNK)])


@jax.jit
def _sc_gather(table, idx):
    mesh = plsc.VectorSubcoreMesh(core_axis_name="c", subcore_axis_name="s")
    return pl.kernel(
        _gather_body,
        out_type=jax.ShapeDtypeStruct((N, D), jnp.float32),
        mesh=mesh,
        scratch_types=[
            pltpu.VMEM((CHUNK,), jnp.int32),
            pltpu.VMEM((CHUNK, D), jnp.float32),
            pltpu.SemaphoreType.DMA,
        ],
        compiler_params=pltpu.CompilerParams(use_tc_tiling_on_sc=False),
    )(table, idx)


def kernel(x, table):
    idx = x.reshape((N,)).astype(jnp.int32)
    out = _sc_gather(table, idx)
    return out.reshape((BATCH, SEQ, D))


# speedup vs baseline: 1.0179x; 1.0179x over previous
import jax
import jax.numpy as jnp
from jax import lax
from jax.experimental import pallas as pl
from jax.experimental.pallas import tpu as pltpu
from jax.experimental.pallas import tpu_sc as plsc

VOCAB = 1000000
D = 64
BATCH = 4096
SEQ = 200
N = BATCH * SEQ

NC = 2
NS = 16
NW = NC * NS
PER_W = N // NW
CHUNK = 400
N_CHUNKS = PER_W // CHUNK
NB = 4
AHEAD = 2


def _gather_body(table_hbm, idx_hbm, out_hbm, idx_v, r0, r1, r2, r3,
                 g0, g1, g2, g3, s0, s1, s2, s3):
    rows = [r0, r1, r2, r3]
    gs = [g0, g1, g2, g3]
    ss = [s0, s1, s2, s3]
    wid = lax.axis_index("s") * NC + lax.axis_index("c")
    w_base = wid * PER_W

    pltpu.sync_copy(idx_hbm.at[pl.ds(w_base, PER_W)], idx_v)

    def idx_slice(j):
        return idx_v.at[pl.ds(j * CHUNK, CHUNK)]

    def out_slice(j):
        return out_hbm.at[pl.ds(w_base + j * CHUNK, CHUNK)]

    for j in range(AHEAD):
        pltpu.async_copy(table_hbm.at[idx_slice(j)], rows[j % NB], gs[j % NB])

    @pl.loop(0, N_CHUNKS, step=NB)
    def _block(i):
        for bb in range(NB):
            j = i + bb
            ga = j + AHEAD
            gb = (bb + AHEAD) % NB

            @pl.when(ga < N_CHUNKS)
            def _issue():
                @pl.when(ga >= NB)
                def _wait_sc():
                    pltpu.make_async_copy(rows[gb], out_slice(ga - NB),
                                          ss[gb]).wait()
                pltpu.async_copy(table_hbm.at[idx_slice(ga)], rows[gb], gs[gb])

            pltpu.make_async_copy(table_hbm.at[idx_slice(j)], rows[bb],
                                  gs[bb]).wait()
            pltpu.async_copy(rows[bb], out_slice(j), ss[bb])

    for bb in range(NB):
        j = N_CHUNKS - NB + bb
        pltpu.make_async_copy(rows[j % NB], out_slice(j), ss[j % NB]).wait()


@jax.jit
def _sc_gather(table, idx):
    mesh = plsc.VectorSubcoreMesh(core_axis_name="c", subcore_axis_name="s")
    return pl.kernel(
        _gather_body,
        out_type=jax.ShapeDtypeStruct((N, D), jnp.float32),
        mesh=mesh,
        scratch_types=(
            [pltpu.VMEM((PER_W,), jnp.int32)]
            + [pltpu.VMEM((CHUNK, D), jnp.float32) for _ in range(NB)]
            + [pltpu.SemaphoreType.DMA for _ in range(2 * NB)]
        ),
        compiler_params=pltpu.CompilerParams(use_tc_tiling_on_sc=False),
    )(table, idx)


def kernel(x, table):
    idx = x.reshape((N,)).astype(jnp.int32)
    out = _sc_gather(table, idx)
    return out.reshape((BATCH, SEQ, D))
